# baseline (speedup 1.0000x reference)
import dataclasses
import functools

import jax
import jax.numpy as jnp
from jax import lax
from jax.experimental import pallas as pl
from jax.experimental.pallas import tpu as pltpu
from jax.experimental.pallas import tpu_sc as plsc

NC = 2
NS = 16
NW = NC * NS
CHUNK = 128
NBUF = 2


def _sc_compiler_params():
    cp = pltpu.CompilerParams()
    if "needs_layout_passes" in pltpu.CompilerParams.__dataclass_fields__:
        cp = dataclasses.replace(cp, needs_layout_passes=False)
    return cp


def _spmm_sc(h, src_p, dst_p, zeros, npad, nchunks):
    d = h.shape[1]
    ept = nchunks * CHUNK
    rpt = npad // NS
    mesh = plsc.VectorSubcoreMesh(core_axis_name="c", subcore_axis_name="s")

    @functools.partial(
        pl.kernel,
        out_type=jax.ShapeDtypeStruct((NC, npad, d), jnp.float32),
        mesh=mesh,
        scratch_types=[
            pltpu.VMEM((CHUNK,), jnp.int32),
            pltpu.VMEM((CHUNK,), jnp.int32),
            pltpu.VMEM((CHUNK, d), jnp.float32),
            pltpu.VMEM_SHARED((npad, d), jnp.float32),
        ],
    )
    def k(h_hbm, src_hbm, dst_hbm, z_hbm, out_hbm, sidx, didx, rows, acc):
        c = lax.axis_index("c")
        s = lax.axis_index("s")
        wid = (1 - c) * NS + s
        pltpu.sync_copy(z_hbm.at[pl.ds(s * rpt, rpt)], acc.at[pl.ds(s * rpt, rpt)])
        plsc.subcore_barrier()
        base = pl.multiple_of(wid * ept, 8)

        @pl.loop(0, nchunks)
        def _(j):
            off = pl.multiple_of(base + j * CHUNK, 8)
            pltpu.sync_copy(src_hbm.at[pl.ds(off, CHUNK)], sidx)
            pltpu.sync_copy(dst_hbm.at[pl.ds(off, CHUNK)], didx)
            pltpu.sync_copy(h_hbm.at[sidx], rows)
            pltpu.sync_copy(rows, acc.at[didx], add=True)

        plsc.subcore_barrier()
        pltpu.sync_copy(acc.at[pl.ds(s * rpt, rpt)],
                        out_hbm.at[c].at[pl.ds(s * rpt, rpt)])

    return k(h, src_p, dst_p, zeros)


def _deg_sc(dst_p, npad, nchunks):
    ept = nchunks * CHUNK
    mesh = plsc.VectorSubcoreMesh(core_axis_name="c", subcore_axis_name="s")

    @functools.partial(
        pl.kernel,
        out_type=jax.ShapeDtypeStruct((NW, npad), jnp.float32),
        mesh=mesh,
        scratch_types=[
            pltpu.VMEM((CHUNK,), jnp.int32),
            pltpu.VMEM((npad,), jnp.float32),
        ],
        compiler_params=_sc_compiler_params(),
    )
    def k(dst_hbm, out_hbm, didx, hist):
        c = lax.axis_index("c")
        s = lax.axis_index("s")
        wid = c * NS + s
        zero_v = jnp.zeros((16,), jnp.float32)
        ones_v = jnp.ones((16,), jnp.float32)

        @pl.loop(0, npad, step=16)
        def _(i):
            hist[pl.ds(i, 16)] = zero_v

        base = pl.multiple_of(wid * ept, 8)

        @pl.loop(0, nchunks)
        def _(j):
            off = pl.multiple_of(base + j * CHUNK, 8)
            pltpu.sync_copy(dst_hbm.at[pl.ds(off, CHUNK)], didx)

            @pl.loop(0, CHUNK, step=16)
            def _(k16):
                idx = didx[pl.ds(k16, 16)]
                plsc.addupdate_scatter(hist, [idx], ones_v)

        pltpu.sync_copy(hist, out_hbm.at[wid])

    return k(dst_p)


def _dense_tc(s_parts, deg_parts, h, w_n, b_n, w_s, b_s, relu, npad):
    d = h.shape[1]

    def body(s_ref, deg_ref, h_ref, wn_ref, bn_ref, ws_ref, bs_ref, o_ref):
        cnt = jnp.sum(deg_ref[...], axis=0)
        cnt = jnp.maximum(cnt, 1.0).reshape(npad, 1)
        agg = (s_ref[0] + s_ref[1]) / cnt
        y = jnp.dot(agg, wn_ref[...], preferred_element_type=jnp.float32)
        y = y + jnp.dot(h_ref[...], ws_ref[...], preferred_element_type=jnp.float32)
        y = y + bn_ref[...] + bs_ref[...]
        if relu:
            y = jnp.maximum(y, 0.0)
        o_ref[...] = y

    return pl.pallas_call(
        body,
        out_shape=jax.ShapeDtypeStruct((npad, d), jnp.float32),
    )(s_parts, deg_parts, h, w_n, b_n, w_s, b_s)


def kernel(x, edge_index, W_neigh1, b_neigh1, W_self1, b_self1,
           W_neigh2, b_neigh2, W_self2, b_self2):
    n, d = x.shape
    e = edge_index.shape[1]
    npad = ((n + NS * 8 - 1) // (NS * 8)) * (NS * 8)
    per_round = NW * CHUNK
    epad = ((e + per_round - 1) // per_round) * per_round
    nchunks = epad // per_round

    src = edge_index[0]
    dst = edge_index[1]
    pad = epad - e
    spare = max(npad - n, 1)
    pad_dst = (n + jnp.arange(pad, dtype=jnp.int32) % spare).astype(jnp.int32)
    pad_dst = jnp.minimum(pad_dst, npad - 1)
    src_p = jnp.concatenate([src, jnp.zeros((pad,), jnp.int32)])
    dst_p = jnp.concatenate([dst, pad_dst])
    x_p = jnp.concatenate([x, jnp.zeros((npad - n, d), jnp.float32)])
    zeros = jnp.zeros((npad, d), jnp.float32)
    bn1 = b_neigh1.reshape(1, d)
    bs1 = b_self1.reshape(1, d)
    bn2 = b_neigh2.reshape(1, d)
    bs2 = b_self2.reshape(1, d)

    deg_parts = _deg_sc(dst_p, npad, nchunks)
    s1 = _spmm_sc(x, src_p, dst_p, zeros, npad, nchunks)
    h1 = _dense_tc(s1, deg_parts, x_p, W_neigh1, bn1, W_self1, bs1, True, npad)
    s2 = _spmm_sc(h1, src_p, dst_p, zeros, npad, nchunks)
    h2 = _dense_tc(s2, deg_parts, h1, W_neigh2, bn2, W_self2, bs2, False, npad)
    return h2[:n]


# speedup vs baseline: 1.5276x; 1.5276x over previous
import dataclasses
import functools

import jax
import jax.numpy as jnp
from jax import lax
from jax.experimental import pallas as pl
from jax.experimental.pallas import tpu as pltpu
from jax.experimental.pallas import tpu_sc as plsc

NC = 2
NS = 16
NW = NC * NS
CHUNK = 128
NBUF = 2


def _sc_compiler_params():
    cp = pltpu.CompilerParams()
    if "needs_layout_passes" in pltpu.CompilerParams.__dataclass_fields__:
        cp = dataclasses.replace(cp, needs_layout_passes=False)
    return cp


def _spmm_sc(h, src_p, dst_p, zeros, npad, nchunks):
    d = h.shape[1]
    ept = nchunks * CHUNK
    rpt = npad // NS
    mesh = plsc.VectorSubcoreMesh(core_axis_name="c", subcore_axis_name="s")

    @functools.partial(
        pl.kernel,
        out_type=jax.ShapeDtypeStruct((NC, npad, d), jnp.float32),
        mesh=mesh,
        scratch_types=[
            pltpu.VMEM((CHUNK,), jnp.int32),
            pltpu.VMEM((CHUNK,), jnp.int32),
            pltpu.VMEM((CHUNK, d), jnp.float32),
            pltpu.VMEM_SHARED((npad, d), jnp.float32),
        ],
    )
    def k(h_hbm, src_hbm, dst_hbm, z_hbm, out_hbm, sidx, didx, rows, acc):
        c = lax.axis_index("c")
        s = lax.axis_index("s")
        wid = c * NS + s
        pltpu.sync_copy(z_hbm.at[pl.ds(s * rpt, rpt)], acc.at[pl.ds(s * rpt, rpt)])
        plsc.subcore_barrier()
        base = pl.multiple_of(wid * ept, 8)

        @pl.loop(0, nchunks)
        def _(j):
            off = pl.multiple_of(base + j * CHUNK, 8)
            pltpu.sync_copy(src_hbm.at[pl.ds(off, CHUNK)], sidx)
            pltpu.sync_copy(dst_hbm.at[pl.ds(off, CHUNK)], didx)
            pltpu.sync_copy(h_hbm.at[sidx], rows)
            pltpu.sync_copy(rows, acc.at[didx], add=True)

        plsc.subcore_barrier()
        pltpu.sync_copy(acc.at[pl.ds(s * rpt, rpt)],
                        out_hbm.at[c].at[pl.ds(s * rpt, rpt)])

    return k(h, src_p, dst_p, zeros)


def _deg_sc(dst_p, npad, nchunks):
    ept = nchunks * CHUNK
    mesh = plsc.VectorSubcoreMesh(core_axis_name="c", subcore_axis_name="s")

    @functools.partial(
        pl.kernel,
        out_type=jax.ShapeDtypeStruct((NW, npad), jnp.float32),
        mesh=mesh,
        scratch_types=[
            pltpu.VMEM((CHUNK,), jnp.int32),
            pltpu.VMEM((npad,), jnp.float32),
        ],
        compiler_params=_sc_compiler_params(),
    )
    def k(dst_hbm, out_hbm, didx, hist):
        c = lax.axis_index("c")
        s = lax.axis_index("s")
        wid = c * NS + s
        zero_v = jnp.zeros((16,), jnp.float32)
        ones_v = jnp.ones((16,), jnp.float32)

        @pl.loop(0, npad, step=16)
        def _(i):
            hist[pl.ds(i, 16)] = zero_v

        base = pl.multiple_of(wid * ept, 8)

        @pl.loop(0, nchunks)
        def _(j):
            off = pl.multiple_of(base + j * CHUNK, 8)
            pltpu.sync_copy(dst_hbm.at[pl.ds(off, CHUNK)], didx)

            @pl.loop(0, CHUNK, step=16)
            def _(k16):
                idx = didx[pl.ds(k16, 16)]
                plsc.addupdate_scatter(hist, [idx], ones_v)

        pltpu.sync_copy(hist, out_hbm.at[wid])

    return k(dst_p)


def _dense_tc(s_parts, deg_parts, h, w_n, b_n, w_s, b_s, relu, npad):
    d = h.shape[1]

    def body(s_ref, deg_ref, h_ref, wn_ref, bn_ref, ws_ref, bs_ref, o_ref):
        cnt = jnp.sum(deg_ref[...], axis=0)
        cnt = jnp.maximum(cnt, 1.0).reshape(npad, 1)
        agg = (s_ref[0] + s_ref[1]) / cnt
        y = jnp.dot(agg, wn_ref[...], preferred_element_type=jnp.float32)
        y = y + jnp.dot(h_ref[...], ws_ref[...], preferred_element_type=jnp.float32)
        y = y + bn_ref[...] + bs_ref[...]
        if relu:
            y = jnp.maximum(y, 0.0)
        o_ref[...] = y

    return pl.pallas_call(
        body,
        out_shape=jax.ShapeDtypeStruct((npad, d), jnp.float32),
    )(s_parts, deg_parts, h, w_n, b_n, w_s, b_s)


def kernel(x, edge_index, W_neigh1, b_neigh1, W_self1, b_self1,
           W_neigh2, b_neigh2, W_self2, b_self2):
    n, d = x.shape
    e = edge_index.shape[1]
    npad = ((n + NS * 8 - 1) // (NS * 8)) * (NS * 8)
    per_round = NW * CHUNK
    epad = ((e + per_round - 1) // per_round) * per_round
    nchunks = epad // per_round

    src = edge_index[0]
    dst = edge_index[1]
    pad = epad - e
    spare = max(npad - n, 1)
    ar = jnp.arange(pad, dtype=jnp.int32)
    pad_dst = jnp.minimum(n + ar % spare, npad - 1).astype(jnp.int32)
    pad_src = (ar * 97) % n
    src_p = jnp.concatenate([src, pad_src])
    dst_p = jnp.concatenate([dst, pad_dst])
    x_p = jnp.concatenate([x, jnp.zeros((npad - n, d), jnp.float32)])
    zeros = jnp.zeros((npad, d), jnp.float32)
    bn1 = b_neigh1.reshape(1, d)
    bs1 = b_self1.reshape(1, d)
    bn2 = b_neigh2.reshape(1, d)
    bs2 = b_self2.reshape(1, d)

    deg_parts = _deg_sc(dst_p, npad, nchunks)
    s1 = _spmm_sc(x, src_p, dst_p, zeros, npad, nchunks)
    h1 = _dense_tc(s1, deg_parts, x_p, W_neigh1, bn1, W_self1, bs1, True, npad)
    s2 = _spmm_sc(h1, src_p, dst_p, zeros, npad, nchunks)
    h2 = _dense_tc(s2, deg_parts, h1, W_neigh2, bn2, W_self2, bs2, False, npad)
    return h2[:n]


# speedup vs baseline: 2.2648x; 1.4826x over previous
import dataclasses
import functools

import jax
import jax.numpy as jnp
from jax import lax
from jax.experimental import pallas as pl
from jax.experimental.pallas import tpu as pltpu
from jax.experimental.pallas import tpu_sc as plsc

NC = 2
NS = 16
NW = NC * NS
CHUNK = 128
NBUF = 2


def _sc_compiler_params():
    cp = pltpu.CompilerParams()
    if "needs_layout_passes" in pltpu.CompilerParams.__dataclass_fields__:
        cp = dataclasses.replace(cp, needs_layout_passes=False)
    return cp


def _spmm_sc(h, src_p, dst_p, zeros, npad, nchunks):
    d = h.shape[1]
    ept = nchunks * CHUNK
    rpt = npad // NS
    mesh = plsc.VectorSubcoreMesh(core_axis_name="c", subcore_axis_name="s")

    @functools.partial(
        pl.kernel,
        out_type=jax.ShapeDtypeStruct((NC, npad, d), jnp.float32),
        mesh=mesh,
        scratch_types=[
            pltpu.VMEM((CHUNK,), jnp.int32),
            pltpu.VMEM((CHUNK,), jnp.int32),
            pltpu.VMEM((CHUNK,), jnp.int32),
            pltpu.VMEM((CHUNK,), jnp.int32),
            pltpu.VMEM((CHUNK, d), jnp.float32),
            pltpu.VMEM((CHUNK, d), jnp.float32),
            pltpu.SemaphoreType.DMA,
            pltpu.SemaphoreType.DMA,
            pltpu.SemaphoreType.DMA,
            pltpu.VMEM_SHARED((npad, d), jnp.float32),
        ],
    )
    def k(h_hbm, src_hbm, dst_hbm, z_hbm, out_hbm, sidx0, didx0, sidx1, didx1,
          rows0, rows1, sa, sg, ss, acc):
        c = lax.axis_index("c")
        s = lax.axis_index("s")
        wid = c * NS + s
        pltpu.sync_copy(z_hbm.at[pl.ds(s * rpt, rpt)], acc.at[pl.ds(s * rpt, rpt)])
        plsc.subcore_barrier()
        base = pl.multiple_of(wid * ept, 8)
        bufs = ((sidx0, didx0, rows0), (sidx1, didx1, rows1))

        @pl.loop(0, nchunks, step=2)
        def _(j):
            ia = []
            for b, (si, di, ro) in enumerate(bufs):
                off = pl.multiple_of(base + (j + b) * CHUNK, 8)
                ia.append(pltpu.async_copy(src_hbm.at[pl.ds(off, CHUNK)], si, sa))
                ia.append(pltpu.async_copy(dst_hbm.at[pl.ds(off, CHUNK)], di, sa))
            for d_ in ia:
                d_.wait()
            g = [pltpu.async_copy(h_hbm.at[si], ro, sg)
                 for (si, di, ro) in bufs]
            sc = []
            for b, (si, di, ro) in enumerate(bufs):
                g[b].wait()
                sc.append(pltpu.async_copy(ro, acc.at[di], ss, add=True))
            for d_ in sc:
                d_.wait()

        plsc.subcore_barrier()
        pltpu.sync_copy(acc.at[pl.ds(s * rpt, rpt)],
                        out_hbm.at[c].at[pl.ds(s * rpt, rpt)])

    return k(h, src_p, dst_p, zeros)


def _deg_sc(dst_p, npad, nchunks):
    ept = nchunks * CHUNK
    mesh = plsc.VectorSubcoreMesh(core_axis_name="c", subcore_axis_name="s")

    @functools.partial(
        pl.kernel,
        out_type=jax.ShapeDtypeStruct((NW, npad), jnp.float32),
        mesh=mesh,
        scratch_types=[
            pltpu.VMEM((CHUNK,), jnp.int32),
            pltpu.VMEM((npad,), jnp.float32),
        ],
        compiler_params=_sc_compiler_params(),
    )
    def k(dst_hbm, out_hbm, didx, hist):
        c = lax.axis_index("c")
        s = lax.axis_index("s")
        wid = c * NS + s
        zero_v = jnp.zeros((16,), jnp.float32)
        ones_v = jnp.ones((16,), jnp.float32)

        @pl.loop(0, npad, step=16)
        def _(i):
            hist[pl.ds(i, 16)] = zero_v

        base = pl.multiple_of(wid * ept, 8)

        @pl.loop(0, nchunks)
        def _(j):
            off = pl.multiple_of(base + j * CHUNK, 8)
            pltpu.sync_copy(dst_hbm.at[pl.ds(off, CHUNK)], didx)

            @pl.loop(0, CHUNK, step=16)
            def _(k16):
                idx = didx[pl.ds(k16, 16)]
                plsc.addupdate_scatter(hist, [idx], ones_v)

        pltpu.sync_copy(hist, out_hbm.at[wid])

    return k(dst_p)


def _dense_tc(s_parts, deg_parts, h, w_n, b_n, w_s, b_s, relu, npad):
    d = h.shape[1]

    def body(s_ref, deg_ref, h_ref, wn_ref, bn_ref, ws_ref, bs_ref, o_ref):
        cnt = jnp.sum(deg_ref[...], axis=0)
        cnt = jnp.maximum(cnt, 1.0).reshape(npad, 1)
        agg = (s_ref[0] + s_ref[1]) / cnt
        y = jnp.dot(agg, wn_ref[...], preferred_element_type=jnp.float32)
        y = y + jnp.dot(h_ref[...], ws_ref[...], preferred_element_type=jnp.float32)
        y = y + bn_ref[...] + bs_ref[...]
        if relu:
            y = jnp.maximum(y, 0.0)
        o_ref[...] = y

    return pl.pallas_call(
        body,
        out_shape=jax.ShapeDtypeStruct((npad, d), jnp.float32),
    )(s_parts, deg_parts, h, w_n, b_n, w_s, b_s)


def kernel(x, edge_index, W_neigh1, b_neigh1, W_self1, b_self1,
           W_neigh2, b_neigh2, W_self2, b_self2):
    n, d = x.shape
    e = edge_index.shape[1]
    npad = ((n + NS * 8 - 1) // (NS * 8)) * (NS * 8)
    per_round = NW * CHUNK * 2
    epad = ((e + per_round - 1) // per_round) * per_round
    nchunks = epad // (NW * CHUNK)

    src = edge_index[0]
    dst = edge_index[1]
    pad = epad - e
    spare = max(npad - n, 1)
    ar = jnp.arange(pad, dtype=jnp.int32)
    pad_dst = jnp.minimum(n + ar % spare, npad - 1).astype(jnp.int32)
    pad_src = (ar * 97) % n
    src_p = jnp.concatenate([src, pad_src])
    dst_p = jnp.concatenate([dst, pad_dst])
    x_p = jnp.concatenate([x, jnp.zeros((npad - n, d), jnp.float32)])
    zeros = jnp.zeros((npad, d), jnp.float32)
    bn1 = b_neigh1.reshape(1, d)
    bs1 = b_self1.reshape(1, d)
    bn2 = b_neigh2.reshape(1, d)
    bs2 = b_self2.reshape(1, d)

    deg_parts = _deg_sc(dst_p, npad, nchunks)
    s1 = _spmm_sc(x, src_p, dst_p, zeros, npad, nchunks)
    h1 = _dense_tc(s1, deg_parts, x_p, W_neigh1, bn1, W_self1, bs1, True, npad)
    s2 = _spmm_sc(h1, src_p, dst_p, zeros, npad, nchunks)
    h2 = _dense_tc(s2, deg_parts, h1, W_neigh2, bn2, W_self2, bs2, False, npad)
    return h2[:n]


# speedup vs baseline: 2.5221x; 1.1136x over previous
import dataclasses
import functools

import jax
import jax.numpy as jnp
from jax import lax
from jax.experimental import pallas as pl
from jax.experimental.pallas import tpu as pltpu
from jax.experimental.pallas import tpu_sc as plsc

NC = 2
NS = 16
NW = NC * NS
CHUNK = 128
NBUF = 2


def _sc_compiler_params():
    cp = pltpu.CompilerParams()
    if "needs_layout_passes" in pltpu.CompilerParams.__dataclass_fields__:
        cp = dataclasses.replace(cp, needs_layout_passes=False)
    return cp


def _spmm_sc(h, src_p, dst_p, zeros, npad, nchunks, with_deg):
    d = h.shape[1]
    ept = nchunks * CHUNK
    rpt = npad // NS
    mesh = plsc.VectorSubcoreMesh(core_axis_name="c", subcore_axis_name="s")
    out_type = [jax.ShapeDtypeStruct((NC, npad, d), jnp.float32)]
    scratch = [
        pltpu.VMEM((CHUNK,), jnp.int32),
        pltpu.VMEM((CHUNK,), jnp.int32),
        pltpu.VMEM((CHUNK,), jnp.int32),
        pltpu.VMEM((CHUNK,), jnp.int32),
        pltpu.VMEM((CHUNK, d), jnp.float32),
        pltpu.VMEM((CHUNK, d), jnp.float32),
        pltpu.SemaphoreType.DMA,
        pltpu.SemaphoreType.DMA,
        pltpu.SemaphoreType.DMA,
        pltpu.VMEM_SHARED((npad, d), jnp.float32),
    ]
    if with_deg:
        out_type.append(jax.ShapeDtypeStruct((NW, npad), jnp.float32))
        scratch.append(pltpu.VMEM((npad,), jnp.float32))

    kw = dict(out_type=out_type, mesh=mesh, scratch_types=scratch)
    if with_deg:
        kw["compiler_params"] = _sc_compiler_params()

    @functools.partial(pl.kernel, **kw)
    def k(h_hbm, src_hbm, dst_hbm, z_hbm, out_hbm, *rest):
        if with_deg:
            deg_hbm, sidx0, didx0, sidx1, didx1, rows0, rows1, sa, sg, ss, acc, hist = rest
        else:
            sidx0, didx0, sidx1, didx1, rows0, rows1, sa, sg, ss, acc = rest
            deg_hbm = hist = None
        c = lax.axis_index("c")
        s = lax.axis_index("s")
        wid = c * NS + s
        pltpu.sync_copy(z_hbm.at[pl.ds(s * rpt, rpt)], acc.at[pl.ds(s * rpt, rpt)])
        if with_deg:
            zero_v = jnp.zeros((16,), jnp.float32)

            @pl.loop(0, npad, step=16)
            def _(i):
                hist[pl.ds(i, 16)] = zero_v

        plsc.subcore_barrier()
        base = pl.multiple_of(wid * ept, 8)
        bufs = ((sidx0, didx0, rows0), (sidx1, didx1, rows1))
        ones_v = jnp.ones((16,), jnp.float32)

        @pl.loop(0, nchunks, step=2)
        def _(j):
            ia = []
            for b, (si, di, ro) in enumerate(bufs):
                off = pl.multiple_of(base + (j + b) * CHUNK, 8)
                ia.append(pltpu.async_copy(src_hbm.at[pl.ds(off, CHUNK)], si, sa))
                ia.append(pltpu.async_copy(dst_hbm.at[pl.ds(off, CHUNK)], di, sa))
            for d_ in ia:
                d_.wait()
            g = [pltpu.async_copy(h_hbm.at[si], ro, sg)
                 for (si, di, ro) in bufs]
            sc = []
            for b, (si, di, ro) in enumerate(bufs):
                g[b].wait()
                sc.append(pltpu.async_copy(ro, acc.at[di], ss, add=True))
                if with_deg:
                    @pl.loop(0, CHUNK, step=16)
                    def _(k16, di=di):
                        plsc.addupdate_scatter(hist, [di[pl.ds(k16, 16)]], ones_v)
            for d_ in sc:
                d_.wait()

        plsc.subcore_barrier()
        pltpu.sync_copy(acc.at[pl.ds(s * rpt, rpt)],
                        out_hbm.at[c].at[pl.ds(s * rpt, rpt)])
        if with_deg:
            pltpu.sync_copy(hist, deg_hbm.at[wid])

    return k(h, src_p, dst_p, zeros)


def _deg_sc(dst_p, npad, nchunks):
    ept = nchunks * CHUNK
    mesh = plsc.VectorSubcoreMesh(core_axis_name="c", subcore_axis_name="s")

    @functools.partial(
        pl.kernel,
        out_type=jax.ShapeDtypeStruct((NW, npad), jnp.float32),
        mesh=mesh,
        scratch_types=[
            pltpu.VMEM((CHUNK,), jnp.int32),
            pltpu.VMEM((npad,), jnp.float32),
        ],
        compiler_params=_sc_compiler_params(),
    )
    def k(dst_hbm, out_hbm, didx, hist):
        c = lax.axis_index("c")
        s = lax.axis_index("s")
        wid = c * NS + s
        zero_v = jnp.zeros((16,), jnp.float32)
        ones_v = jnp.ones((16,), jnp.float32)

        @pl.loop(0, npad, step=16)
        def _(i):
            hist[pl.ds(i, 16)] = zero_v

        base = pl.multiple_of(wid * ept, 8)

        @pl.loop(0, nchunks)
        def _(j):
            off = pl.multiple_of(base + j * CHUNK, 8)
            pltpu.sync_copy(dst_hbm.at[pl.ds(off, CHUNK)], didx)

            @pl.loop(0, CHUNK, step=16)
            def _(k16):
                idx = didx[pl.ds(k16, 16)]
                plsc.addupdate_scatter(hist, [idx], ones_v)

        pltpu.sync_copy(hist, out_hbm.at[wid])

    return k(dst_p)


def _dense_tc(s_parts, deg_parts, h, w_n, b_n, w_s, b_s, relu, npad):
    d = h.shape[1]

    def body(s_ref, deg_ref, h_ref, wn_ref, bn_ref, ws_ref, bs_ref, o_ref):
        cnt = jnp.sum(deg_ref[...], axis=0)
        cnt = jnp.maximum(cnt, 1.0).reshape(npad, 1)
        agg = (s_ref[0] + s_ref[1]) / cnt
        y = jnp.dot(agg, wn_ref[...], preferred_element_type=jnp.float32)
        y = y + jnp.dot(h_ref[...], ws_ref[...], preferred_element_type=jnp.float32)
        y = y + bn_ref[...] + bs_ref[...]
        if relu:
            y = jnp.maximum(y, 0.0)
        o_ref[...] = y

    return pl.pallas_call(
        body,
        out_shape=jax.ShapeDtypeStruct((npad, d), jnp.float32),
    )(s_parts, deg_parts, h, w_n, b_n, w_s, b_s)


def kernel(x, edge_index, W_neigh1, b_neigh1, W_self1, b_self1,
           W_neigh2, b_neigh2, W_self2, b_self2):
    n, d = x.shape
    e = edge_index.shape[1]
    npad = ((n + NS * 8 - 1) // (NS * 8)) * (NS * 8)
    per_round = NW * CHUNK * 2
    epad = ((e + per_round - 1) // per_round) * per_round
    nchunks = epad // (NW * CHUNK)

    src = edge_index[0]
    dst = edge_index[1]
    pad = epad - e
    spare = max(npad - n, 1)
    ar = jnp.arange(pad, dtype=jnp.int32)
    pad_dst = jnp.minimum(n + ar % spare, npad - 1).astype(jnp.int32)
    pad_src = (ar * 97) % n
    src_p = jnp.concatenate([src, pad_src])
    dst_p = jnp.concatenate([dst, pad_dst])
    x_p = jnp.concatenate([x, jnp.zeros((npad - n, d), jnp.float32)])
    zeros = jnp.zeros((npad, d), jnp.float32)
    bn1 = b_neigh1.reshape(1, d)
    bs1 = b_self1.reshape(1, d)
    bn2 = b_neigh2.reshape(1, d)
    bs2 = b_self2.reshape(1, d)

    s1, deg_parts = _spmm_sc(x, src_p, dst_p, zeros, npad, nchunks, True)
    h1 = _dense_tc(s1, deg_parts, x_p, W_neigh1, bn1, W_self1, bs1, True, npad)
    (s2,) = _spmm_sc(h1, src_p, dst_p, zeros, npad, nchunks, False)
    h2 = _dense_tc(s2, deg_parts, h1, W_neigh2, bn2, W_self2, bs2, False, npad)
    return h2[:n]
